# baseline (speedup 1.0000x reference)
import jax
import jax.numpy as jnp
from jax.experimental import pallas as pl

_T = 0.05
_BQ = 512
_BK = 4096


def _matmul_kernel(x_ref, m_ref, o_ref):
    xb = (x_ref[...] * (1.0 / _T)).astype(jnp.bfloat16)
    mb = m_ref[...].astype(jnp.bfloat16)
    o_ref[...] = jnp.dot(xb, mb.T, preferred_element_type=jnp.float32)


def kernel(x, mem_feat):
    q, d = x.shape
    k = mem_feat.shape[0]
    grid = (k // _BK, q // _BQ)
    return pl.pallas_call(
        _matmul_kernel,
        grid=grid,
        in_specs=[
            pl.BlockSpec((_BQ, d), lambda i, j: (j, 0)),
            pl.BlockSpec((_BK, d), lambda i, j: (i, 0)),
        ],
        out_specs=pl.BlockSpec((_BQ, _BK), lambda i, j: (j, i)),
        out_shape=jax.ShapeDtypeStruct((q, k), jnp.float32),
    )(x, mem_feat)


# speedup vs baseline: 1.1093x; 1.1093x over previous
import jax
import jax.numpy as jnp
from jax.experimental import pallas as pl

_T = 0.05
_BK = 4096


def _matmul_kernel(x_ref, m_ref, o_ref):
    xb = (x_ref[...] * (1.0 / _T)).astype(jnp.bfloat16)
    mb = m_ref[...].astype(jnp.bfloat16)
    o_ref[...] = jnp.dot(xb, mb.T, preferred_element_type=jnp.float32)


def kernel(x, mem_feat):
    q, d = x.shape
    k = mem_feat.shape[0]
    grid = (k // _BK,)
    return pl.pallas_call(
        _matmul_kernel,
        grid=grid,
        in_specs=[
            pl.BlockSpec((q, d), lambda i: (0, 0)),
            pl.BlockSpec((_BK, d), lambda i: (i, 0)),
        ],
        out_specs=pl.BlockSpec((q, _BK), lambda i: (0, i)),
        out_shape=jax.ShapeDtypeStruct((q, k), jnp.float32),
    )(x, mem_feat)
